# baseline (speedup 1.0000x reference)
import functools

import jax
import jax.numpy as jnp
from jax import lax
from jax.experimental import pallas as pl
from jax.experimental.pallas import tpu as pltpu
from jax.experimental.pallas import tpu_sc as plsc

N_SECTOR = 100000
EMBED_DIM = 32

NUM_CORES = 2
NUM_SUBCORES = 16
NUM_WORKERS = NUM_CORES * NUM_SUBCORES

CHUNK = 1664


def _make_kernel(B: int):
    assert B % (NUM_WORKERS * CHUNK) == 0
    b_per_w = B // NUM_WORKERS
    n_chunks = b_per_w // CHUNK

    mesh = plsc.VectorSubcoreMesh(
        core_axis_name="c", subcore_axis_name="s", num_cores=NUM_CORES
    )

    @functools.partial(
        pl.kernel,
        mesh=mesh,
        out_type=jax.ShapeDtypeStruct((B, EMBED_DIM), jnp.float32),
        scratch_types=[
            pltpu.VMEM((n_chunks, CHUNK), jnp.int32),
            pltpu.VMEM((2, CHUNK, EMBED_DIM), jnp.float32),
            pltpu.SemaphoreType.DMA,
            pltpu.SemaphoreType.DMA,
            pltpu.SemaphoreType.DMA,
            pltpu.SemaphoreType.DMA,
        ],
        compiler_params=pltpu.CompilerParams(use_tc_tiling_on_sc=False),
    )
    def k(table_hbm, idx_hbm, out_hbm, idx_v, rows_v, g0, g1, w0, w1):
        wid = lax.axis_index("s") * NUM_CORES + lax.axis_index("c")
        base = wid * b_per_w
        pltpu.sync_copy(idx_hbm.at[wid], idx_v)

        gsem = (g0, g1)
        wsem = (w0, w1)

        def fire_gather(j, b):
            return pltpu.async_copy(
                table_hbm.at[idx_v.at[j]], rows_v.at[b], gsem[b]
            )

        def fire_write(j, b):
            return pltpu.async_copy(
                rows_v.at[b], out_hbm.at[pl.ds(base + j * CHUNK, CHUNK)], wsem[b]
            )

        gathers = [None, None]
        writes = [None, None]
        gathers[0] = fire_gather(0, 0)
        for j in range(n_chunks):
            b = j & 1
            ob = b ^ 1
            gathers[b].wait()
            if j + 1 < n_chunks:
                if writes[ob] is not None:
                    writes[ob].wait()
                gathers[ob] = fire_gather(j + 1, ob)
            writes[b] = fire_write(j, b)
        writes[(n_chunks - 1) & 1].wait()
        if n_chunks > 1:
            writes[n_chunks & 1].wait()

    return k


def kernel(x, table):
    B_rows, F = x.shape
    B = B_rows * F
    idx = x.reshape(NUM_WORKERS, -1, CHUNK).astype(jnp.int32)
    out = _make_kernel(B)(table, idx)
    return out.reshape(B_rows, F, EMBED_DIM)
